# baseline (speedup 1.0000x reference)
import functools

import jax
import jax.numpy as jnp
from jax import lax
from jax.experimental import pallas as pl
from jax.experimental.pallas import tpu as pltpu
from jax.experimental.pallas import tpu_sc as plsc

B = 4096
F = 26
V = 100000
E = 32
OUT = 128
CUST = 128

NC = 2
NS = 16
NW = NC * NS
N_PER_W = (B * F) // NW


FIELD_CHUNKS = (7, 7, 6, 6)


@functools.lru_cache(maxsize=None)
def _make_gather(nf):
    mesh = plsc.VectorSubcoreMesh(core_axis_name="c", subcore_axis_name="s")
    n_per_w = (B * nf) // NW

    @functools.partial(
        pl.kernel,
        mesh=mesh,
        out_type=jax.ShapeDtypeStruct((B * nf, E), jnp.float32),
        scratch_types=[
            pltpu.VMEM((n_per_w,), jnp.int32),
            pltpu.VMEM((n_per_w, E), jnp.float32),
            pltpu.SemaphoreType.DMA,
        ],
        compiler_params=pltpu.CompilerParams(use_tc_tiling_on_sc=False),
    )
    def gather_k(idx_hbm, table_hbm, out_hbm, idx_v, rows_v, sem):
        wid = lax.axis_index("s") * NC + lax.axis_index("c")
        base = wid * n_per_w
        pltpu.sync_copy(idx_hbm.at[pl.ds(base, n_per_w)], idx_v)
        pltpu.async_copy(table_hbm.at[idx_v], rows_v, sem).wait()
        pltpu.sync_copy(rows_v, out_hbm.at[pl.ds(base, n_per_w)])

    return gather_k


def _dense_body(emb_ref, cust_ref, wa_ref, ba_ref, wc1_ref, bc1_ref, wc2_ref,
                bc2_ref, out_ref):
    a = jnp.dot(emb_ref[...], wa_ref[...], preferred_element_type=jnp.float32)
    a = jnp.maximum(a + ba_ref[...], 0.0)
    c = jnp.dot(cust_ref[...], wc1_ref[...], preferred_element_type=jnp.float32)
    c = jnp.maximum(c + bc1_ref[...], 0.0)
    c = jnp.dot(c, wc2_ref[...], preferred_element_type=jnp.float32) + bc2_ref[...]
    logits = jnp.sum(a * c, axis=1)
    out_ref[...] = 1.0 / (1.0 + jnp.exp(-logits))


CBLK = 4000


def _compact_body(t_ref, o_ref):
    xb = t_ref[...]
    q = CBLK // 4
    o_ref[...] = jnp.concatenate(
        [xb[0:q], xb[q:2 * q], xb[2 * q:3 * q], xb[3 * q:4 * q]], axis=1)


@functools.lru_cache(maxsize=None)
def _make_compact(nf, o, interpret=False):
    base = o * V // CBLK
    return pl.pallas_call(
        _compact_body,
        grid=(nf * V // CBLK,),
        in_specs=[pl.BlockSpec((CBLK, E), lambda i: (base + i, 0))],
        out_specs=pl.BlockSpec((CBLK // 4, 4 * E), lambda i: (i, 0)),
        out_shape=jax.ShapeDtypeStruct((nf * V // 4, 4 * E), jnp.float32),
        interpret=interpret,
    )


BLK = 512


def _make_dense(interpret=False):
    grid = (B // BLK,)
    return pl.pallas_call(
        _dense_body,
        grid=grid,
        in_specs=[
            pl.BlockSpec((BLK, F * E), lambda i: (i, 0)),
            pl.BlockSpec((BLK, CUST), lambda i: (i, 0)),
            pl.BlockSpec((F * E, OUT), lambda i: (0, 0)),
            pl.BlockSpec((1, OUT), lambda i: (0, 0)),
            pl.BlockSpec((CUST, 8), lambda i: (0, 0)),
            pl.BlockSpec((1, 8), lambda i: (0, 0)),
            pl.BlockSpec((8, OUT), lambda i: (0, 0)),
            pl.BlockSpec((1, OUT), lambda i: (0, 0)),
        ],
        out_specs=pl.BlockSpec((BLK,), lambda i: (i,)),
        out_shape=jax.ShapeDtypeStruct((B,), jnp.float32),
        interpret=interpret,
    )


_dense = _make_dense()


def kernel(customer_features, article_features, tables, W_a, b_a, W_c1, b_c1,
           W_c2, b_c2):
    idx = article_features.astype(jnp.int32)
    emb_parts = []
    o = 0
    for nf in FIELD_CHUNKS:
        sub = idx[:, o:o + nf] + (jnp.arange(nf, dtype=jnp.int32) * V)[None, :]
        flat_tab = _make_compact(nf, o)(
            tables.reshape(F * V, E)).reshape(nf * V, E)
        blk = sub // CBLK
        w = sub % CBLK
        pos = 4 * (blk * (CBLK // 4) + w % (CBLK // 4)) + w // (CBLK // 4)
        rows = _make_gather(nf)(pos.reshape(-1), flat_tab)
        emb_parts.append(rows.reshape(B, nf * E))
        o += nf
    emb = jnp.concatenate(emb_parts, axis=1)

    wc1 = jnp.pad(W_c1, ((0, 0), (0, 3)))
    bc1 = jnp.pad(b_c1, (0, 3)).reshape(1, 8)
    wc2 = jnp.pad(W_c2, ((0, 3), (0, 0)))

    return _dense(emb, customer_features, W_a, b_a.reshape(1, OUT), wc1, bc1,
                  wc2, b_c2.reshape(1, OUT))


# speedup vs baseline: 1.2304x; 1.2304x over previous
import functools

import jax
import jax.numpy as jnp
from jax import lax
from jax.experimental import pallas as pl
from jax.experimental.pallas import tpu as pltpu
from jax.experimental.pallas import tpu_sc as plsc

B = 4096
F = 26
V = 100000
E = 32
OUT = 128
CUST = 128

NC = 2
NS = 16
NW = NC * NS
N_PER_W = (B * F) // NW


FIELD_CHUNKS = (26,)


@functools.lru_cache(maxsize=None)
def _make_gather(nf):
    mesh = plsc.VectorSubcoreMesh(core_axis_name="c", subcore_axis_name="s")
    n_per_w = (B * nf) // NW

    @functools.partial(
        pl.kernel,
        mesh=mesh,
        out_type=jax.ShapeDtypeStruct((B * nf, E), jnp.float32),
        scratch_types=[
            pltpu.VMEM((n_per_w,), jnp.int32),
            pltpu.VMEM((n_per_w, E), jnp.float32),
            pltpu.SemaphoreType.DMA,
        ],
        compiler_params=pltpu.CompilerParams(use_tc_tiling_on_sc=False),
    )
    def gather_k(idx_hbm, table_hbm, out_hbm, idx_v, rows_v, sem):
        wid = lax.axis_index("s") * NC + lax.axis_index("c")
        base = wid * n_per_w
        pltpu.sync_copy(idx_hbm.at[pl.ds(base, n_per_w)], idx_v)
        pltpu.async_copy(table_hbm.at[idx_v], rows_v, sem).wait()
        pltpu.sync_copy(rows_v, out_hbm.at[pl.ds(base, n_per_w)])

    return gather_k


def _dense_body(emb_ref, cust_ref, wa_ref, ba_ref, wc1_ref, bc1_ref, wc2_ref,
                bc2_ref, out_ref):
    a = jnp.dot(emb_ref[...], wa_ref[...], preferred_element_type=jnp.float32)
    a = jnp.maximum(a + ba_ref[...], 0.0)
    c = jnp.dot(cust_ref[...], wc1_ref[...], preferred_element_type=jnp.float32)
    c = jnp.maximum(c + bc1_ref[...], 0.0)
    c = jnp.dot(c, wc2_ref[...], preferred_element_type=jnp.float32) + bc2_ref[...]
    logits = jnp.sum(a * c, axis=1)
    out_ref[...] = 1.0 / (1.0 + jnp.exp(-logits))


CBLK = 8000


def _compact_body(t_ref, o_ref):
    xb = t_ref[...]
    q = CBLK // 4
    o_ref[...] = jnp.concatenate(
        [xb[0:q], xb[q:2 * q], xb[2 * q:3 * q], xb[3 * q:4 * q]], axis=1)


@functools.lru_cache(maxsize=None)
def _make_compact(nf, o, interpret=False):
    base = o * V // CBLK
    return pl.pallas_call(
        _compact_body,
        grid=(nf * V // CBLK,),
        in_specs=[pl.BlockSpec((CBLK, E), lambda i: (base + i, 0))],
        out_specs=pl.BlockSpec((CBLK // 4, 4 * E), lambda i: (i, 0)),
        out_shape=jax.ShapeDtypeStruct((nf * V // 4, 4 * E), jnp.float32),
        interpret=interpret,
    )


BLK = 512


def _make_dense(interpret=False):
    grid = (B // BLK,)
    return pl.pallas_call(
        _dense_body,
        grid=grid,
        in_specs=[
            pl.BlockSpec((BLK, F * E), lambda i: (i, 0)),
            pl.BlockSpec((BLK, CUST), lambda i: (i, 0)),
            pl.BlockSpec((F * E, OUT), lambda i: (0, 0)),
            pl.BlockSpec((1, OUT), lambda i: (0, 0)),
            pl.BlockSpec((CUST, 8), lambda i: (0, 0)),
            pl.BlockSpec((1, 8), lambda i: (0, 0)),
            pl.BlockSpec((8, OUT), lambda i: (0, 0)),
            pl.BlockSpec((1, OUT), lambda i: (0, 0)),
        ],
        out_specs=pl.BlockSpec((BLK,), lambda i: (i,)),
        out_shape=jax.ShapeDtypeStruct((B,), jnp.float32),
        interpret=interpret,
    )


_dense = _make_dense()


def kernel(customer_features, article_features, tables, W_a, b_a, W_c1, b_c1,
           W_c2, b_c2):
    idx = article_features.astype(jnp.int32)
    emb_parts = []
    o = 0
    for nf in FIELD_CHUNKS:
        sub = idx[:, o:o + nf] + (jnp.arange(nf, dtype=jnp.int32) * V)[None, :]
        flat_tab = _make_compact(nf, o)(
            tables.reshape(F * V, E)).reshape(nf * V, E)
        blk = sub // CBLK
        w = sub % CBLK
        pos = 4 * (blk * (CBLK // 4) + w % (CBLK // 4)) + w // (CBLK // 4)
        rows = _make_gather(nf)(pos.reshape(-1), flat_tab)
        emb_parts.append(rows.reshape(B, nf * E))
        o += nf
    emb = jnp.concatenate(emb_parts, axis=1)

    wc1 = jnp.pad(W_c1, ((0, 0), (0, 3)))
    bc1 = jnp.pad(b_c1, (0, 3)).reshape(1, 8)
    wc2 = jnp.pad(W_c2, ((0, 3), (0, 0)))

    return _dense(emb, customer_features, W_a, b_a.reshape(1, OUT), wc1, bc1,
                  wc2, b_c2.reshape(1, OUT))


# speedup vs baseline: 1.2917x; 1.0498x over previous
import functools

import jax
import jax.numpy as jnp
from jax import lax
from jax.experimental import pallas as pl
from jax.experimental.pallas import tpu as pltpu
from jax.experimental.pallas import tpu_sc as plsc

B = 4096
F = 26
V = 100000
E = 32
OUT = 128
CUST = 128

NC = 2
NS = 16
NW = NC * NS
N_PER_W = (B * F) // NW


FIELD_CHUNKS = (26,)


@functools.lru_cache(maxsize=None)
def _make_gather(nf):
    mesh = plsc.VectorSubcoreMesh(core_axis_name="c", subcore_axis_name="s")
    n_per_w = (B * nf) // NW

    @functools.partial(
        pl.kernel,
        mesh=mesh,
        out_type=jax.ShapeDtypeStruct((B * nf, E), jnp.float32),
        scratch_types=[
            pltpu.VMEM((n_per_w,), jnp.int32),
            pltpu.VMEM((n_per_w, E), jnp.float32),
            pltpu.SemaphoreType.DMA,
        ],
        compiler_params=pltpu.CompilerParams(use_tc_tiling_on_sc=False),
    )
    def gather_k(idx_hbm, table_hbm, out_hbm, idx_v, rows_v, sem):
        wid = lax.axis_index("s") * NC + lax.axis_index("c")
        base = wid * n_per_w
        pltpu.sync_copy(idx_hbm.at[pl.ds(base, n_per_w)], idx_v)
        pltpu.async_copy(table_hbm.at[idx_v], rows_v, sem).wait()
        pltpu.sync_copy(rows_v, out_hbm.at[pl.ds(base, n_per_w)])

    return gather_k


def _dense_body(emb_ref, cust_ref, wa_ref, ba_ref, wc1_ref, bc1_ref, wc2_ref,
                bc2_ref, out_ref):
    a = jnp.dot(emb_ref[...], wa_ref[...], preferred_element_type=jnp.float32)
    a = jnp.maximum(a + ba_ref[...], 0.0)
    c = jnp.dot(cust_ref[...], wc1_ref[...], preferred_element_type=jnp.float32)
    c = jnp.maximum(c + bc1_ref[...], 0.0)
    c = jnp.dot(c, wc2_ref[...], preferred_element_type=jnp.float32) + bc2_ref[...]
    logits = jnp.sum(a * c, axis=1)
    out_ref[...] = 1.0 / (1.0 + jnp.exp(-logits))


CBLK = 10400


def _compact_body(t_ref, o_ref):
    xb = t_ref[...]
    q = CBLK // 4
    o_ref[...] = jnp.concatenate(
        [xb[0:q], xb[q:2 * q], xb[2 * q:3 * q], xb[3 * q:4 * q]], axis=1)


@functools.lru_cache(maxsize=None)
def _make_compact(nf, o, interpret=False):
    base = o * V // CBLK
    return pl.pallas_call(
        _compact_body,
        grid=(nf * V // CBLK,),
        in_specs=[pl.BlockSpec((CBLK, E), lambda i: (base + i, 0))],
        out_specs=pl.BlockSpec((CBLK // 4, 4 * E), lambda i: (i, 0)),
        out_shape=jax.ShapeDtypeStruct((nf * V // 4, 4 * E), jnp.float32),
        interpret=interpret,
    )


BLK = 512


def _make_dense(interpret=False):
    grid = (B // BLK,)
    return pl.pallas_call(
        _dense_body,
        grid=grid,
        in_specs=[
            pl.BlockSpec((BLK, F * E), lambda i: (i, 0)),
            pl.BlockSpec((BLK, CUST), lambda i: (i, 0)),
            pl.BlockSpec((F * E, OUT), lambda i: (0, 0)),
            pl.BlockSpec((1, OUT), lambda i: (0, 0)),
            pl.BlockSpec((CUST, 8), lambda i: (0, 0)),
            pl.BlockSpec((1, 8), lambda i: (0, 0)),
            pl.BlockSpec((8, OUT), lambda i: (0, 0)),
            pl.BlockSpec((1, OUT), lambda i: (0, 0)),
        ],
        out_specs=pl.BlockSpec((BLK,), lambda i: (i,)),
        out_shape=jax.ShapeDtypeStruct((B,), jnp.float32),
        interpret=interpret,
    )


_dense = _make_dense()


def kernel(customer_features, article_features, tables, W_a, b_a, W_c1, b_c1,
           W_c2, b_c2):
    idx = article_features.astype(jnp.int32)
    emb_parts = []
    o = 0
    for nf in FIELD_CHUNKS:
        sub = idx[:, o:o + nf] + (jnp.arange(nf, dtype=jnp.int32) * V)[None, :]
        flat_tab = _make_compact(nf, o)(
            tables.reshape(F * V, E)).reshape(nf * V, E)
        blk = sub // CBLK
        w = sub % CBLK
        pos = 4 * (blk * (CBLK // 4) + w % (CBLK // 4)) + w // (CBLK // 4)
        rows = _make_gather(nf)(pos.reshape(-1), flat_tab)
        emb_parts.append(rows.reshape(B, nf * E))
        o += nf
    emb = jnp.concatenate(emb_parts, axis=1)

    wc1 = jnp.pad(W_c1, ((0, 0), (0, 3)))
    bc1 = jnp.pad(b_c1, (0, 3)).reshape(1, 8)
    wc2 = jnp.pad(W_c2, ((0, 3), (0, 0)))

    return _dense(emb, customer_features, W_a, b_a.reshape(1, OUT), wc1, bc1,
                  wc2, b_c2.reshape(1, OUT))


# speedup vs baseline: 1.3511x; 1.0460x over previous
import functools

import jax
import jax.numpy as jnp
from jax import lax
from jax.experimental import pallas as pl
from jax.experimental.pallas import tpu as pltpu
from jax.experimental.pallas import tpu_sc as plsc

B = 4096
F = 26
V = 100000
E = 32
OUT = 128
CUST = 128

NC = 2
NS = 16
NW = NC * NS
N_PER_W = (B * F) // NW


FIELD_CHUNKS = (26,)


@functools.lru_cache(maxsize=None)
def _make_gather(nf):
    mesh = plsc.VectorSubcoreMesh(core_axis_name="c", subcore_axis_name="s")
    n_per_w = (B * nf) // NW

    @functools.partial(
        pl.kernel,
        mesh=mesh,
        out_type=jax.ShapeDtypeStruct((B * nf, E), jnp.float32),
        scratch_types=[
            pltpu.VMEM((n_per_w,), jnp.int32),
            pltpu.VMEM((n_per_w, E), jnp.float32),
            pltpu.SemaphoreType.DMA,
        ],
        compiler_params=pltpu.CompilerParams(use_tc_tiling_on_sc=False),
    )
    def gather_k(idx_hbm, table_hbm, out_hbm, idx_v, rows_v, sem):
        wid = lax.axis_index("s") * NC + lax.axis_index("c")
        base = wid * n_per_w
        pltpu.sync_copy(idx_hbm.at[pl.ds(base, n_per_w)], idx_v)
        pltpu.async_copy(table_hbm.at[idx_v], rows_v, sem).wait()
        pltpu.sync_copy(rows_v, out_hbm.at[pl.ds(base, n_per_w)])

    return gather_k


def _dense_body(emb_ref, cust_ref, wa_ref, ba_ref, wc1_ref, bc1_ref, wc2_ref,
                bc2_ref, out_ref):
    a = jnp.dot(emb_ref[...], wa_ref[...], preferred_element_type=jnp.float32)
    a = jnp.maximum(a + ba_ref[...], 0.0)
    c = jnp.dot(cust_ref[...], wc1_ref[...], preferred_element_type=jnp.float32)
    c = jnp.maximum(c + bc1_ref[...], 0.0)
    c = jnp.dot(c, wc2_ref[...], preferred_element_type=jnp.float32) + bc2_ref[...]
    logits = jnp.sum(a * c, axis=1)
    out_ref[...] = 1.0 / (1.0 + jnp.exp(-logits))


CBLK = 20800


def _compact_body(t_ref, o_ref):
    xb = t_ref[...]
    q = CBLK // 4
    o_ref[...] = jnp.concatenate(
        [xb[0:q], xb[q:2 * q], xb[2 * q:3 * q], xb[3 * q:4 * q]], axis=1)


@functools.lru_cache(maxsize=None)
def _make_compact(nf, o, interpret=False):
    base = o * V // CBLK
    return pl.pallas_call(
        _compact_body,
        grid=(nf * V // CBLK,),
        in_specs=[pl.BlockSpec((CBLK, E), lambda i: (base + i, 0))],
        out_specs=pl.BlockSpec((CBLK // 4, 4 * E), lambda i: (i, 0)),
        out_shape=jax.ShapeDtypeStruct((nf * V // 4, 4 * E), jnp.float32),
        interpret=interpret,
    )


BLK = 512


def _make_dense(interpret=False):
    grid = (B // BLK,)
    return pl.pallas_call(
        _dense_body,
        grid=grid,
        in_specs=[
            pl.BlockSpec((BLK, F * E), lambda i: (i, 0)),
            pl.BlockSpec((BLK, CUST), lambda i: (i, 0)),
            pl.BlockSpec((F * E, OUT), lambda i: (0, 0)),
            pl.BlockSpec((1, OUT), lambda i: (0, 0)),
            pl.BlockSpec((CUST, 8), lambda i: (0, 0)),
            pl.BlockSpec((1, 8), lambda i: (0, 0)),
            pl.BlockSpec((8, OUT), lambda i: (0, 0)),
            pl.BlockSpec((1, OUT), lambda i: (0, 0)),
        ],
        out_specs=pl.BlockSpec((BLK,), lambda i: (i,)),
        out_shape=jax.ShapeDtypeStruct((B,), jnp.float32),
        interpret=interpret,
    )


_dense = _make_dense()


def kernel(customer_features, article_features, tables, W_a, b_a, W_c1, b_c1,
           W_c2, b_c2):
    idx = article_features.astype(jnp.int32)
    emb_parts = []
    o = 0
    for nf in FIELD_CHUNKS:
        sub = idx[:, o:o + nf] + (jnp.arange(nf, dtype=jnp.int32) * V)[None, :]
        flat_tab = _make_compact(nf, o)(
            tables.reshape(F * V, E)).reshape(nf * V, E)
        blk = sub // CBLK
        w = sub % CBLK
        pos = 4 * (blk * (CBLK // 4) + w % (CBLK // 4)) + w // (CBLK // 4)
        rows = _make_gather(nf)(pos.reshape(-1), flat_tab)
        emb_parts.append(rows.reshape(B, nf * E))
        o += nf
    emb = jnp.concatenate(emb_parts, axis=1)

    wc1 = jnp.pad(W_c1, ((0, 0), (0, 3)))
    bc1 = jnp.pad(b_c1, (0, 3)).reshape(1, 8)
    wc2 = jnp.pad(W_c2, ((0, 3), (0, 0)))

    return _dense(emb, customer_features, W_a, b_a.reshape(1, OUT), wc1, bc1,
                  wc2, b_c2.reshape(1, OUT))


# speedup vs baseline: 1.3563x; 1.0039x over previous
import functools

import jax
import jax.numpy as jnp
from jax import lax
from jax.experimental import pallas as pl
from jax.experimental.pallas import tpu as pltpu
from jax.experimental.pallas import tpu_sc as plsc

B = 4096
F = 26
V = 100000
E = 32
OUT = 128
CUST = 128

NC = 2
NS = 16
NW = NC * NS
N_PER_W = (B * F) // NW


FIELD_CHUNKS = (26,)


@functools.lru_cache(maxsize=None)
def _make_gather(nf):
    mesh = plsc.VectorSubcoreMesh(core_axis_name="c", subcore_axis_name="s")
    n_per_w = (B * nf) // NW

    @functools.partial(
        pl.kernel,
        mesh=mesh,
        out_type=jax.ShapeDtypeStruct((B * nf, E), jnp.float32),
        scratch_types=[
            pltpu.VMEM((n_per_w,), jnp.int32),
            pltpu.VMEM((n_per_w, E), jnp.float32),
            pltpu.SemaphoreType.DMA,
        ],
        compiler_params=pltpu.CompilerParams(use_tc_tiling_on_sc=False),
    )
    def gather_k(idx_hbm, table_hbm, out_hbm, idx_v, rows_v, sem):
        wid = lax.axis_index("s") * NC + lax.axis_index("c")
        base = wid * n_per_w
        pltpu.sync_copy(idx_hbm.at[pl.ds(base, n_per_w)], idx_v)
        pltpu.async_copy(table_hbm.at[idx_v], rows_v, sem).wait()
        pltpu.sync_copy(rows_v, out_hbm.at[pl.ds(base, n_per_w)])

    return gather_k


def _dense_body(emb_ref, cust_ref, wa_ref, ba_ref, wc1_ref, bc1_ref, wc2_ref,
                bc2_ref, out_ref):
    a = jnp.dot(emb_ref[...], wa_ref[...], preferred_element_type=jnp.float32)
    a = jnp.maximum(a + ba_ref[...], 0.0)
    c = jnp.dot(cust_ref[...], wc1_ref[...], preferred_element_type=jnp.float32)
    c = jnp.maximum(c + bc1_ref[...], 0.0)
    c = jnp.dot(c, wc2_ref[...], preferred_element_type=jnp.float32) + bc2_ref[...]
    logits = jnp.sum(a * c, axis=1)
    out_ref[...] = 1.0 / (1.0 + jnp.exp(-logits))


CBLK = 40000


def _compact_body(t_ref, o_ref):
    xb = t_ref[...]
    q = CBLK // 4
    o_ref[...] = jnp.concatenate(
        [xb[0:q], xb[q:2 * q], xb[2 * q:3 * q], xb[3 * q:4 * q]], axis=1)


@functools.lru_cache(maxsize=None)
def _make_compact(nf, o, interpret=False):
    base = o * V // CBLK
    return pl.pallas_call(
        _compact_body,
        grid=(nf * V // CBLK,),
        in_specs=[pl.BlockSpec((CBLK, E), lambda i: (base + i, 0))],
        out_specs=pl.BlockSpec((CBLK // 4, 4 * E), lambda i: (i, 0)),
        out_shape=jax.ShapeDtypeStruct((nf * V // 4, 4 * E), jnp.float32),
        interpret=interpret,
    )


BLK = 512


def _make_dense(interpret=False):
    grid = (B // BLK,)
    return pl.pallas_call(
        _dense_body,
        grid=grid,
        in_specs=[
            pl.BlockSpec((BLK, F * E), lambda i: (i, 0)),
            pl.BlockSpec((BLK, CUST), lambda i: (i, 0)),
            pl.BlockSpec((F * E, OUT), lambda i: (0, 0)),
            pl.BlockSpec((1, OUT), lambda i: (0, 0)),
            pl.BlockSpec((CUST, 8), lambda i: (0, 0)),
            pl.BlockSpec((1, 8), lambda i: (0, 0)),
            pl.BlockSpec((8, OUT), lambda i: (0, 0)),
            pl.BlockSpec((1, OUT), lambda i: (0, 0)),
        ],
        out_specs=pl.BlockSpec((BLK,), lambda i: (i,)),
        out_shape=jax.ShapeDtypeStruct((B,), jnp.float32),
        interpret=interpret,
    )


_dense = _make_dense()


def kernel(customer_features, article_features, tables, W_a, b_a, W_c1, b_c1,
           W_c2, b_c2):
    idx = article_features.astype(jnp.int32)
    emb_parts = []
    o = 0
    for nf in FIELD_CHUNKS:
        sub = idx[:, o:o + nf] + (jnp.arange(nf, dtype=jnp.int32) * V)[None, :]
        flat_tab = _make_compact(nf, o)(
            tables.reshape(F * V, E)).reshape(nf * V, E)
        blk = sub // CBLK
        w = sub % CBLK
        pos = 4 * (blk * (CBLK // 4) + w % (CBLK // 4)) + w // (CBLK // 4)
        rows = _make_gather(nf)(pos.reshape(-1), flat_tab)
        emb_parts.append(rows.reshape(B, nf * E))
        o += nf
    emb = jnp.concatenate(emb_parts, axis=1)

    wc1 = jnp.pad(W_c1, ((0, 0), (0, 3)))
    bc1 = jnp.pad(b_c1, (0, 3)).reshape(1, 8)
    wc2 = jnp.pad(W_c2, ((0, 3), (0, 0)))

    return _dense(emb, customer_features, W_a, b_a.reshape(1, OUT), wc1, bc1,
                  wc2, b_c2.reshape(1, OUT))


# speedup vs baseline: 1.3573x; 1.0007x over previous
import functools

import jax
import jax.numpy as jnp
from jax import lax
from jax.experimental import pallas as pl
from jax.experimental.pallas import tpu as pltpu
from jax.experimental.pallas import tpu_sc as plsc

B = 4096
F = 26
V = 100000
E = 32
OUT = 128
CUST = 128

NC = 2
NS = 16
NW = NC * NS

FIELD_CHUNKS = (26,)


@functools.lru_cache(maxsize=None)
def _make_gather(nf):
    mesh = plsc.VectorSubcoreMesh(core_axis_name="c", subcore_axis_name="s")
    n_per_w = (B * nf) // NW

    @functools.partial(
        pl.kernel,
        mesh=mesh,
        out_type=jax.ShapeDtypeStruct((B * nf, E), jnp.float32),
        scratch_types=[
            pltpu.VMEM((n_per_w,), jnp.int32),
            pltpu.VMEM((n_per_w, E), jnp.float32),
            pltpu.SemaphoreType.DMA,
        ],
        compiler_params=pltpu.CompilerParams(use_tc_tiling_on_sc=False),
    )
    def gather_k(idx_hbm, table_hbm, out_hbm, idx_v, rows_v, sem):
        wid = lax.axis_index("s") * NC + lax.axis_index("c")
        base = wid * n_per_w
        pltpu.sync_copy(idx_hbm.at[pl.ds(base, n_per_w)], idx_v)
        pltpu.async_copy(table_hbm.at[idx_v], rows_v, sem).wait()
        pltpu.sync_copy(rows_v, out_hbm.at[pl.ds(base, n_per_w)])

    return gather_k


def _dense_body(emb_ref, cust_ref, wa_ref, ba_ref, wc1_ref, bc1_ref, wc2_ref,
                bc2_ref, out_ref):
    a = jnp.dot(emb_ref[...], wa_ref[...], preferred_element_type=jnp.float32)
    a = jnp.maximum(a + ba_ref[...], 0.0)
    c = jnp.dot(cust_ref[...], wc1_ref[...], preferred_element_type=jnp.float32)
    c = jnp.maximum(c + bc1_ref[...], 0.0)
    c = jnp.dot(c, wc2_ref[...], preferred_element_type=jnp.float32) + bc2_ref[...]
    logits = jnp.sum(a * c, axis=1)
    out_ref[...] = 1.0 / (1.0 + jnp.exp(-logits))


CBLK = 40000


def _compact_body(t_ref, o_ref):
    xb = t_ref[...]
    q = CBLK // 4
    o_ref[...] = jnp.concatenate(
        [xb[0:q], xb[q:2 * q], xb[2 * q:3 * q], xb[3 * q:4 * q]], axis=1)


@functools.lru_cache(maxsize=None)
def _make_compact(nf, o, interpret=False):
    base = o * V // CBLK
    return pl.pallas_call(
        _compact_body,
        grid=(nf * V // CBLK,),
        in_specs=[pl.BlockSpec((CBLK, E), lambda i: (base + i, 0))],
        out_specs=pl.BlockSpec((CBLK // 4, 4 * E), lambda i: (i, 0)),
        out_shape=jax.ShapeDtypeStruct((nf * V // 4, 4 * E), jnp.float32),
        interpret=interpret,
    )


BLK = 512


def _make_dense(interpret=False):
    grid = (B // BLK,)
    return pl.pallas_call(
        _dense_body,
        grid=grid,
        in_specs=[
            pl.BlockSpec((BLK, F * E), lambda i: (i, 0)),
            pl.BlockSpec((BLK, CUST), lambda i: (i, 0)),
            pl.BlockSpec((F * E, OUT), lambda i: (0, 0)),
            pl.BlockSpec((1, OUT), lambda i: (0, 0)),
            pl.BlockSpec((CUST, 8), lambda i: (0, 0)),
            pl.BlockSpec((1, 8), lambda i: (0, 0)),
            pl.BlockSpec((8, OUT), lambda i: (0, 0)),
            pl.BlockSpec((1, OUT), lambda i: (0, 0)),
        ],
        out_specs=pl.BlockSpec((BLK,), lambda i: (i,)),
        out_shape=jax.ShapeDtypeStruct((B,), jnp.float32),
        interpret=interpret,
    )


_dense = _make_dense()


def kernel(customer_features, article_features, tables, W_a, b_a, W_c1, b_c1,
           W_c2, b_c2):
    idx = article_features.astype(jnp.int32)
    emb_parts = []
    o = 0
    for nf in FIELD_CHUNKS:
        sub = idx[:, o:o + nf] + (jnp.arange(nf, dtype=jnp.int32) * V)[None, :]
        flat_tab = _make_compact(nf, o)(
            tables.reshape(F * V, E)).reshape(nf * V, E)
        blk = sub // CBLK
        w = sub % CBLK
        pos = 4 * (blk * (CBLK // 4) + w % (CBLK // 4)) + w // (CBLK // 4)
        rows = _make_gather(nf)(pos.reshape(-1), flat_tab)
        emb_parts.append(rows.reshape(B, nf * E))
        o += nf
    emb = jnp.concatenate(emb_parts, axis=1)

    wc1 = jnp.pad(W_c1, ((0, 0), (0, 3)))
    bc1 = jnp.pad(b_c1, (0, 3)).reshape(1, 8)
    wc2 = jnp.pad(W_c2, ((0, 3), (0, 0)))

    return _dense(emb, customer_features, W_a, b_a.reshape(1, OUT), wc1, bc1,
                  wc2, b_c2.reshape(1, OUT))
